# baseline (speedup 1.0000x reference)
import functools

import jax
import jax.numpy as jnp
from jax import lax
from jax.experimental import pallas as pl
from jax.experimental.pallas import tpu as pltpu
from jax.experimental.pallas import tpu_sc as plsc

NC, NS = 2, 16
NW = NC * NS
B, L, D = 16384, 50, 32
LP = 128
BPW = B // NW
HPW = BPW // 2
BR = 16
NBLK_H = HPW // BR
LG = 56
HALF = 16
INV_L = 1.0 / L

_mesh = plsc.VectorSubcoreMesh(
    core_axis_name="c", subcore_axis_name="s", num_cores=NC, num_subcores=NS
)


@functools.partial(
    pl.kernel,
    out_type=jax.ShapeDtypeStruct((B, D), jnp.float32),
    mesh=_mesh,
    scratch_types=[
        pltpu.VMEM((HPW, LP), jnp.int32),
        pltpu.VMEM((2, BR, LG, D), jnp.float32),
        pltpu.VMEM((BPW, D), jnp.float32),
        pltpu.SemaphoreType.DMA,
        pltpu.SemaphoreType.DMA,
    ],
    compiler_params=pltpu.CompilerParams(use_tc_tiling_on_sc=False),
)
def _emb_lookup_mean(table_hbm, idx_hbm, out_hbm, idx_v, rows_v, out_v,
                     sem0, sem1):
    sems = (sem0, sem1)
    wid = lax.axis_index("s") * NC + lax.axis_index("c")

    def fire(p, blk):
        for r in range(BR):
            pltpu.async_copy(
                table_hbm.at[idx_v.at[blk * BR + r, pl.ds(0, LG)]],
                rows_v.at[p, r],
                sems[p],
            )

    def drain(p):
        for r in range(BR):
            pltpu.make_async_copy(
                table_hbm.at[idx_v.at[r, pl.ds(0, LG)]],
                rows_v.at[p, r],
                sems[p],
            ).wait()

    def accum(p, blk, obase):
        def row(r, carry):
            acc0 = rows_v[p, r, 0, 0:HALF]
            acc1 = rows_v[p, r, 0, HALF:D]
            for l in range(1, L):
                acc0 = acc0 + rows_v[p, r, l, 0:HALF]
                acc1 = acc1 + rows_v[p, r, l, HALF:D]
            orow = obase + blk * BR + r
            out_v[orow, 0:HALF] = acc0 * INV_L
            out_v[orow, HALF:D] = acc1 * INV_L
            return carry

        lax.fori_loop(0, BR, row, 0)

    for h in range(2):
        obase = h * HPW
        pltpu.sync_copy(idx_hbm.at[pl.ds(wid * BPW + obase, HPW)], idx_v)
        fire(0, 0)

        def body(g2, carry, obase=obase):
            ga = 2 * g2
            fire(1, ga + 1)
            drain(0)
            accum(0, ga, obase)
            fire(0, ga + 2)
            drain(1)
            accum(1, ga + 1, obase)
            return carry

        lax.fori_loop(0, NBLK_H // 2 - 1, body, 0)

        fire(1, NBLK_H - 1)
        drain(0)
        accum(0, NBLK_H - 2, obase)
        drain(1)
        accum(1, NBLK_H - 1, obase)

    pltpu.sync_copy(out_v, out_hbm.at[pl.ds(wid * BPW, BPW)])


def kernel(inputs, table):
    idx = jnp.pad(inputs.astype(jnp.int32), ((0, 0), (0, LP - L)))
    return _emb_lookup_mean(table, idx)


# speedup vs baseline: 2.6553x; 2.6553x over previous
import functools

import jax
import jax.numpy as jnp
from jax import lax
from jax.experimental import pallas as pl
from jax.experimental.pallas import tpu as pltpu
from jax.experimental.pallas import tpu_sc as plsc

NC, NS = 2, 16
NW = NC * NS
B, L, D = 16384, 50, 32
NROWS = 1000001
BPW = B // NW
BR = 16
NBLK = BPW // BR
HALF = 16
INV_L = 1.0 / L

_mesh = plsc.VectorSubcoreMesh(
    core_axis_name="c", subcore_axis_name="s", num_cores=NC, num_subcores=NS
)


@functools.partial(
    pl.kernel,
    out_type=jax.ShapeDtypeStruct((B, D), jnp.float32),
    mesh=_mesh,
    scratch_types=[
        pltpu.VMEM((BPW, L), jnp.int32),
        pltpu.VMEM((2, BR, L, D), jnp.float32),
        pltpu.VMEM((BPW, D), jnp.float32),
        pltpu.SemaphoreType.DMA,
        pltpu.SemaphoreType.DMA,
    ],
    compiler_params=pltpu.CompilerParams(use_tc_tiling_on_sc=False),
)
def _emb_lookup_mean(table_hbm, idx_hbm, out_hbm, idx_v, rows_v, out_v,
                     sem0, sem1):
    sems = (sem0, sem1)
    wid = lax.axis_index("s") * NC + lax.axis_index("c")
    pltpu.sync_copy(idx_hbm.at[pl.ds(wid * BPW, BPW)], idx_v)

    def fire(p, blk):
        for r in range(BR):
            pltpu.async_copy(
                table_hbm.at[idx_v.at[blk * BR + r, :]],
                rows_v.at[p, r],
                sems[p],
            )

    def drain(p):
        for r in range(BR):
            pltpu.make_async_copy(
                table_hbm.at[idx_v.at[r, :]],
                rows_v.at[p, r],
                sems[p],
            ).wait()

    def accum(p, blk):
        def row(r, carry):
            acc0 = rows_v[p, r, 0, 0:HALF]
            acc1 = rows_v[p, r, 0, HALF:D]
            for l in range(1, L):
                acc0 = acc0 + rows_v[p, r, l, 0:HALF]
                acc1 = acc1 + rows_v[p, r, l, HALF:D]
            orow = blk * BR + r
            out_v[orow, 0:HALF] = acc0 * INV_L
            out_v[orow, HALF:D] = acc1 * INV_L
            return carry

        lax.fori_loop(0, BR, row, 0)

    fire(0, 0)

    def body(g2, carry):
        ga = 2 * g2
        fire(1, ga + 1)
        drain(0)
        accum(0, ga)
        fire(0, ga + 2)
        drain(1)
        accum(1, ga + 1)
        return carry

    lax.fori_loop(0, NBLK // 2 - 1, body, 0)

    fire(1, NBLK - 1)
    drain(0)
    accum(0, NBLK - 2)
    drain(1)
    accum(1, NBLK - 1)

    pltpu.sync_copy(out_v, out_hbm.at[pl.ds(wid * BPW, BPW)])


def kernel(inputs, table):
    flat = lax.optimization_barrier(table.reshape(NROWS * D))
    return _emb_lookup_mean(flat.reshape(NROWS, D), inputs)
